# Initial kernel scaffold; baseline (speedup 1.0000x reference)
import jax
import jax.numpy as jnp
from jax.experimental import pallas as pl


def kernel(x_drug_v1, x_dis_v1, x_drug_v2, x_dis_v2, ei_dd_v1, ei_rev_v1, ei_dd_v2, ei_rev_v2, W_rel_dd, W_rel_rev, Ws_drug, Ws_dis, Wp1_drug, bp1_drug, Wp2_drug, bp2_drug, Wp1_dis, bp1_dis, Wp2_dis, bp2_dis):
    raise NotImplementedError("write your pallas kernel here")


# speedup vs baseline: 1.0815x; 1.0815x over previous
import jax
import jax.numpy as jnp
from jax import lax
from jax.experimental import pallas as pl
from jax.experimental.pallas import tpu as pltpu
from jax.experimental.pallas import tpu_sc as plsc

_N = 50000
_E = 250000
_D = 128
_NC, _NS = 2, 16
_NPASS = 2
_RNG = 12544
_NPAD = _NC * _NPASS * _RNG
_ACC = _RNG + 16
_ZPT = _ACC // _NS
_WPT = _RNG // _NS
_CH = 128
_NCHUNK = -(-(_E // _NS) // _CH)
_EPT = _NCHUNK * _CH
_EPAD = _EPT * _NS
_BM = 512
_GRID = _NPAD // _BM


def _make_segsum():
  mesh = plsc.VectorSubcoreMesh(core_axis_name="c", subcore_axis_name="s")
  out_type = [jax.ShapeDtypeStruct((_NPAD, _D), jnp.float32)] * 4
  scratch = [
      pltpu.VMEM_SHARED((_ACC, _D), jnp.float32),
      pltpu.VMEM((_CH,), jnp.int32),
      pltpu.VMEM((_CH,), jnp.int32),
      pltpu.VMEM((_CH,), jnp.int32),
      pltpu.VMEM((_CH, _D), jnp.float32),
  ]

  def body(*refs):
    hs = refs[0:4]
    srcs = refs[4:8]
    dsts = refs[8:12]
    Ss = refs[12:16]
    acc, src_v, dst_v, dloc_v, rows_v = refs[16:]
    cid = lax.axis_index("c")
    sid = lax.axis_index("s")
    zero16 = jnp.zeros((16,), jnp.float32)

    for r in range(4):
      h, se, de, S = hs[r], srcs[r], dsts[r], Ss[r]

      def pass_body(p, carry, h=h, se=se, de=de, S=S):
        base = (cid * _NPASS + p) * _RNG

        def fill(i, fcarry):
          for j in range(_D // 16):
            rows_v[i, pl.ds(16 * j, 16)] = zero16
          return fcarry

        lax.fori_loop(0, _CH, fill, 0)

        def zloop(zc, zcarry):
          off = jnp.minimum(zc * _CH, _ZPT - _CH)
          row0 = sid * _ZPT + off
          pltpu.sync_copy(rows_v, acc.at[pl.ds(row0, _CH)])
          return zcarry

        lax.fori_loop(0, -(-_ZPT // _CH), zloop, 0)
        plsc.subcore_barrier()

        def cloop(ch, ccarry):
          e0 = sid * _EPT + ch * _CH
          pltpu.sync_copy(se.at[pl.ds(e0, _CH)], src_v)
          pltpu.sync_copy(de.at[pl.ds(e0, _CH)], dst_v)

          def jloop(j, jcarry):
            d = dst_v[pl.ds(j * 16, 16)]
            m = (d >= base) & (d < base + _RNG)
            dloc_v[pl.ds(j * 16, 16)] = jnp.where(m, d - base, _RNG)
            return jcarry

          lax.fori_loop(0, _CH // 16, jloop, 0)
          pltpu.sync_copy(h.at[src_v], rows_v)
          pltpu.sync_copy(rows_v, acc.at[dloc_v], add=True)
          return ccarry

        lax.fori_loop(0, _NCHUNK, cloop, 0)
        plsc.subcore_barrier()
        row0 = sid * _WPT
        pltpu.sync_copy(acc.at[pl.ds(row0, _WPT)],
                        S.at[pl.ds(base + row0, _WPT)])
        plsc.subcore_barrier()
        return carry

      lax.fori_loop(0, _NPASS, pass_body, 0)

  return pl.kernel(body, out_type=tuple(out_type), mesh=mesh,
                   scratch_types=tuple(scratch))


def _make_counts():
  mesh = plsc.VectorSubcoreMesh(core_axis_name="c", subcore_axis_name="s")
  out_type = [jax.ShapeDtypeStruct((_NPAD, 16), jnp.float32)] * 4
  scratch = [
      pltpu.VMEM_SHARED((_ACC, 16), jnp.float32),
      pltpu.VMEM((_CH,), jnp.int32),
      pltpu.VMEM((_CH,), jnp.int32),
      pltpu.VMEM((_CH, 16), jnp.float32),
      pltpu.VMEM((_CH, 16), jnp.float32),
  ]

  def body(*refs):
    dsts = refs[0:4]
    Cs = refs[4:8]
    accc, dst_v, dloc_v, ones_v, zbuf16 = refs[8:]
    cid = lax.axis_index("c")
    sid = lax.axis_index("s")
    zero16 = jnp.zeros((16,), jnp.float32)
    one16 = jnp.ones((16,), jnp.float32)

    def fill(i, carry):
      ones_v[i, :] = one16
      zbuf16[i, :] = zero16
      return carry

    lax.fori_loop(0, _CH, fill, 0)

    for r in range(4):
      de, C = dsts[r], Cs[r]

      def pass_body(p, carry, de=de, C=C):
        base = (cid * _NPASS + p) * _RNG

        def zloop(zc, zcarry):
          off = jnp.minimum(zc * _CH, _ZPT - _CH)
          pltpu.sync_copy(zbuf16, accc.at[pl.ds(sid * _ZPT + off, _CH)])
          return zcarry

        lax.fori_loop(0, -(-_ZPT // _CH), zloop, 0)
        plsc.subcore_barrier()

        def cloop(ch, ccarry):
          e0 = sid * _EPT + ch * _CH
          pltpu.sync_copy(de.at[pl.ds(e0, _CH)], dst_v)

          def jloop(j, jcarry):
            d = dst_v[pl.ds(j * 16, 16)]
            m = (d >= base) & (d < base + _RNG)
            dloc_v[pl.ds(j * 16, 16)] = jnp.where(m, d - base, _RNG)
            return jcarry

          lax.fori_loop(0, _CH // 16, jloop, 0)
          pltpu.sync_copy(ones_v, accc.at[dloc_v], add=True)
          return ccarry

        lax.fori_loop(0, _NCHUNK, cloop, 0)
        plsc.subcore_barrier()
        row0 = sid * _WPT
        pltpu.sync_copy(accc.at[pl.ds(row0, _WPT)],
                        C.at[pl.ds(base + row0, _WPT)])
        plsc.subcore_barrier()
        return carry

      lax.fori_loop(0, _NPASS, pass_body, 0)

  return pl.kernel(body, out_type=tuple(out_type), mesh=mesh,
                   scratch_types=tuple(scratch))


def _dense_update(x, s, c, ws, wr):
  def body(x_ref, s_ref, c_ref, ws_ref, wr_ref, o_ref):
    r = 1.0 / jnp.maximum(c_ref[:, 0:1], 1.0)
    acc = jnp.dot(x_ref[:], ws_ref[:], preferred_element_type=jnp.float32)
    acc = acc + jnp.dot(s_ref[:], wr_ref[:],
                        preferred_element_type=jnp.float32) * r
    o_ref[:] = jnp.maximum(acc, 0.0)

  return pl.pallas_call(
      body,
      grid=(_GRID,),
      in_specs=[
          pl.BlockSpec((_BM, _D), lambda i: (i, 0)),
          pl.BlockSpec((_BM, _D), lambda i: (i, 0)),
          pl.BlockSpec((_BM, 16), lambda i: (i, 0)),
          pl.BlockSpec((_D, _D), lambda i: (0, 0)),
          pl.BlockSpec((_D, _D), lambda i: (0, 0)),
      ],
      out_specs=pl.BlockSpec((_BM, _D), lambda i: (i, 0)),
      out_shape=jax.ShapeDtypeStruct((_NPAD, _D), jnp.float32),
  )(x, s, c, ws, wr)


def _project(h, w1, b1, w2, b2):
  ph, pd = w1.shape[1], w2.shape[1]

  def body(h_ref, w1_ref, b1_ref, w2_ref, b2_ref, o_ref):
    t = jnp.dot(h_ref[:], w1_ref[:], preferred_element_type=jnp.float32)
    t = jax.nn.gelu(t + b1_ref[:])
    o_ref[:] = jnp.dot(t, w2_ref[:],
                       preferred_element_type=jnp.float32) + b2_ref[:]

  return pl.pallas_call(
      body,
      grid=(_GRID,),
      in_specs=[
          pl.BlockSpec((_BM, _D), lambda i: (i, 0)),
          pl.BlockSpec((_D, ph), lambda i: (0, 0)),
          pl.BlockSpec((1, ph), lambda i: (0, 0)),
          pl.BlockSpec((ph, pd), lambda i: (0, 0)),
          pl.BlockSpec((1, pd), lambda i: (0, 0)),
      ],
      out_specs=pl.BlockSpec((_BM, pd), lambda i: (i, 0)),
      out_shape=jax.ShapeDtypeStruct((_NPAD, pd), jnp.float32),
  )(h, w1, b1.reshape(1, ph), w2, b2.reshape(1, pd))


def kernel(x_drug_v1, x_dis_v1, x_drug_v2, x_dis_v2,
           ei_dd_v1, ei_rev_v1, ei_dd_v2, ei_rev_v2,
           W_rel_dd, W_rel_rev, Ws_drug, Ws_dis,
           Wp1_drug, bp1_drug, Wp2_drug, bp2_drug,
           Wp1_dis, bp1_dis, Wp2_dis, bp2_dis):
  pad_n = lambda x: jnp.pad(x, ((0, _NPAD - _N), (0, 0)))
  hd1, hs1, hd2, hs2 = map(pad_n, (x_drug_v1, x_dis_v1, x_drug_v2, x_dis_v2))

  def pad_e(ei):
    src = jnp.pad(ei[0].astype(jnp.int32), (0, _EPAD - _E))
    dst = jnp.pad(ei[1].astype(jnp.int32), (0, _EPAD - _E),
                  constant_values=_N)
    return src, dst

  s_dd1, d_dd1 = pad_e(ei_dd_v1)
  s_rv1, d_rv1 = pad_e(ei_rev_v1)
  s_dd2, d_dd2 = pad_e(ei_dd_v2)
  s_rv2, d_rv2 = pad_e(ei_rev_v2)

  seg = _make_segsum()
  cntk = _make_counts()

  C_dd1, C_rv1, C_dd2, C_rv2 = cntk(d_dd1, d_rv1, d_dd2, d_rv2)
  S_dd1, S_rv1, S_dd2, S_rv2 = seg(
      hd1, hs1, hd2, hs2,
      s_dd1, s_rv1, s_dd2, s_rv2,
      d_dd1, d_rv1, d_dd2, d_rv2)
  hd1n = _dense_update(hd1, S_rv1, C_rv1, Ws_drug[0], W_rel_rev[0])
  hs1n = _dense_update(hs1, S_dd1, C_dd1, Ws_dis[0], W_rel_dd[0])
  hd2n = _dense_update(hd2, S_rv2, C_rv2, Ws_drug[0], W_rel_rev[0])
  hs2n = _dense_update(hs2, S_dd2, C_dd2, Ws_dis[0], W_rel_dd[0])

  S_dd1, S_rv1, S_dd2, S_rv2 = seg(
      hd1n, hs1n, hd2n, hs2n,
      s_dd1, s_rv1, s_dd2, s_rv2,
      d_dd1, d_rv1, d_dd2, d_rv2)
  hd1f = _dense_update(hd1n, S_rv1, C_rv1, Ws_drug[1], W_rel_rev[1])
  hs1f = _dense_update(hs1n, S_dd1, C_dd1, Ws_dis[1], W_rel_dd[1])
  hd2f = _dense_update(hd2n, S_rv2, C_rv2, Ws_drug[1], W_rel_rev[1])
  hs2f = _dense_update(hs2n, S_dd2, C_dd2, Ws_dis[1], W_rel_dd[1])

  p1d = _project(hd1f, Wp1_drug, bp1_drug, Wp2_drug, bp2_drug)[:_N]
  p1s = _project(hs1f, Wp1_dis, bp1_dis, Wp2_dis, bp2_dis)[:_N]
  p2d = _project(hd2f, Wp1_drug, bp1_drug, Wp2_drug, bp2_drug)[:_N]
  p2s = _project(hs2f, Wp1_dis, bp1_dis, Wp2_dis, bp2_dis)[:_N]
  return (p1d, p1s, p2d, p2s)


# speedup vs baseline: 1.1601x; 1.0726x over previous
import jax
import jax.numpy as jnp
from jax import lax
from jax.experimental import pallas as pl
from jax.experimental.pallas import tpu as pltpu
from jax.experimental.pallas import tpu_sc as plsc

_N = 50000
_E = 250000
_D = 128
_NC, _NS = 2, 16
_NPASS = 2
_RNG = 12544
_NPAD = _NC * _NPASS * _RNG
_ACC = _RNG + 16
_ZPT = _ACC // _NS
_WPT = _RNG // _NS
_CH = 128
_NCHUNK = 123
_EPT = _NCHUNK * _CH
_CH2 = _CH
_NCHUNK2 = _NCHUNK
_EPT2 = _EPT
_EPAD = _NS * _EPT
_BM = 512
_GRID = _NPAD // _BM


def _make_segsum():
  mesh = plsc.VectorSubcoreMesh(core_axis_name="c", subcore_axis_name="s")
  out_type = [jax.ShapeDtypeStruct((_NPAD, _D), jnp.float32)] * 4
  scratch = [
      pltpu.VMEM_SHARED((_ACC, _D), jnp.float32),
      pltpu.VMEM((_CH,), jnp.int32), pltpu.VMEM((_CH,), jnp.int32),
      pltpu.VMEM((_CH,), jnp.int32), pltpu.VMEM((_CH,), jnp.int32),
      pltpu.VMEM((_CH,), jnp.int32), pltpu.VMEM((_CH,), jnp.int32),
      pltpu.VMEM((_CH, _D), jnp.float32),
      pltpu.SemaphoreType.DMA, pltpu.SemaphoreType.DMA,
  ]

  def body(*refs):
    hs = refs[0:4]
    srcs = refs[4:8]
    dsts = refs[8:12]
    Ss = refs[12:16]
    (acc, src0, src1, dst0, dst1, dl0, dl1, rows0,
     sia0, sib0) = refs[16:]
    cid = lax.axis_index("c")
    sid = lax.axis_index("s")
    zero16 = jnp.zeros((16,), jnp.float32)

    def compute_dloc(dst_r, dl_r, base):
      def jloop(j, jcarry):
        d = dst_r[pl.ds(j * 16, 16)]
        m = (d >= base) & (d < base + _RNG)
        dl_r[pl.ds(j * 16, 16)] = jnp.where(m, d - base, _RNG)
        return jcarry
      lax.fori_loop(0, _CH // 16, jloop, 0)

    for r in range(4):
      h, se, de, S = hs[r], srcs[r], dsts[r], Ss[r]

      def pass_body(p, carry, h=h, se=se, de=de, S=S):
        base = (cid * _NPASS + p) * _RNG
        ebase = sid * _EPT

        def fill(i, fcarry):
          for j in range(_D // 16):
            rows0[i, pl.ds(16 * j, 16)] = zero16
          return fcarry

        lax.fori_loop(0, _CH, fill, 0)

        def zloop(zc, zcarry):
          off = jnp.minimum(zc * _CH, _ZPT - _CH)
          pltpu.sync_copy(rows0, acc.at[pl.ds(sid * _ZPT + off, _CH)])
          return zcarry

        lax.fori_loop(0, -(-_ZPT // _CH), zloop, 0)
        plsc.subcore_barrier()

        def cloop(ch, ccarry):
          e0 = ebase + ch * _CH
          ia = pltpu.async_copy(se.at[pl.ds(e0, _CH)], src0, sia0)
          ib = pltpu.async_copy(de.at[pl.ds(e0, _CH)], dst0, sib0)
          ib.wait()
          compute_dloc(dst0, dl0, base)
          ia.wait()
          pltpu.sync_copy(h.at[src0], rows0)
          pltpu.sync_copy(rows0, acc.at[dl0], add=True)
          return ccarry

        lax.fori_loop(0, _NCHUNK, cloop, 0)
        plsc.subcore_barrier()
        row0 = sid * _WPT
        pltpu.sync_copy(acc.at[pl.ds(row0, _WPT)],
                        S.at[pl.ds(base + row0, _WPT)])
        plsc.subcore_barrier()
        return carry

      lax.fori_loop(0, _NPASS, pass_body, 0)

  return pl.kernel(body, out_type=tuple(out_type), mesh=mesh,
                   scratch_types=tuple(scratch))


def _make_counts():
  mesh = plsc.VectorSubcoreMesh(core_axis_name="c", subcore_axis_name="s")
  out_type = [jax.ShapeDtypeStruct((_NPAD, 16), jnp.float32)] * 4
  scratch = [
      pltpu.VMEM_SHARED((_ACC, 16), jnp.float32),
      pltpu.VMEM((_CH2,), jnp.int32), pltpu.VMEM((_CH2,), jnp.int32),
      pltpu.VMEM((_CH2,), jnp.int32), pltpu.VMEM((_CH2,), jnp.int32),
      pltpu.VMEM((_CH2, 16), jnp.float32),
      pltpu.VMEM((_CH2, 16), jnp.float32),
      pltpu.SemaphoreType.DMA, pltpu.SemaphoreType.DMA,
  ]

  def body(*refs):
    dsts = refs[0:4]
    Cs = refs[4:8]
    accc, dst0, dst1, dl0, dl1, ones_v, zbuf16, si0, si1 = refs[8:]
    cid = lax.axis_index("c")
    sid = lax.axis_index("s")
    zero16 = jnp.zeros((16,), jnp.float32)
    one16 = jnp.ones((16,), jnp.float32)

    def fill(i, carry):
      ones_v[i, :] = one16
      zbuf16[i, :] = zero16
      return carry

    lax.fori_loop(0, _CH2, fill, 0)

    def compute_dloc(dst_r, dl_r, base):
      def jloop(j, jcarry):
        d = dst_r[pl.ds(j * 16, 16)]
        m = (d >= base) & (d < base + _RNG)
        dl_r[pl.ds(j * 16, 16)] = jnp.where(m, d - base, _RNG)
        return jcarry
      lax.fori_loop(0, _CH2 // 16, jloop, 0)

    for r in range(4):
      de, C = dsts[r], Cs[r]

      def pass_body(p, carry, de=de, C=C):
        base = (cid * _NPASS + p) * _RNG
        ebase = sid * _EPT2

        def zloop(zc, zcarry):
          off = jnp.minimum(zc * _CH2, _ZPT - _CH2)
          pltpu.sync_copy(zbuf16, accc.at[pl.ds(sid * _ZPT + off, _CH2)])
          return zcarry

        lax.fori_loop(0, -(-_ZPT // _CH2), zloop, 0)
        plsc.subcore_barrier()

        def cloop(ch, ccarry):
          e0 = ebase + ch * _CH2
          pltpu.sync_copy(de.at[pl.ds(e0, _CH2)], dst0)
          compute_dloc(dst0, dl0, base)
          pltpu.sync_copy(ones_v, accc.at[dl0], add=True)
          return ccarry

        lax.fori_loop(0, _NCHUNK2, cloop, 0)
        plsc.subcore_barrier()
        row0 = sid * _WPT
        pltpu.sync_copy(accc.at[pl.ds(row0, _WPT)],
                        C.at[pl.ds(base + row0, _WPT)])
        plsc.subcore_barrier()
        return carry

      lax.fori_loop(0, _NPASS, pass_body, 0)

  return pl.kernel(body, out_type=tuple(out_type), mesh=mesh,
                   scratch_types=tuple(scratch))


def _dense_update(x, s, c, ws, wr):
  def body(x_ref, s_ref, c_ref, ws_ref, wr_ref, o_ref):
    r = 1.0 / jnp.maximum(c_ref[:, 0:1], 1.0)
    acc = jnp.dot(x_ref[:], ws_ref[:], preferred_element_type=jnp.float32)
    acc = acc + jnp.dot(s_ref[:], wr_ref[:],
                        preferred_element_type=jnp.float32) * r
    o_ref[:] = jnp.maximum(acc, 0.0)

  return pl.pallas_call(
      body,
      grid=(_GRID,),
      in_specs=[
          pl.BlockSpec((_BM, _D), lambda i: (i, 0)),
          pl.BlockSpec((_BM, _D), lambda i: (i, 0)),
          pl.BlockSpec((_BM, 16), lambda i: (i, 0)),
          pl.BlockSpec((_D, _D), lambda i: (0, 0)),
          pl.BlockSpec((_D, _D), lambda i: (0, 0)),
      ],
      out_specs=pl.BlockSpec((_BM, _D), lambda i: (i, 0)),
      out_shape=jax.ShapeDtypeStruct((_NPAD, _D), jnp.float32),
  )(x, s, c, ws, wr)


def _project(h, w1, b1, w2, b2):
  ph, pd = w1.shape[1], w2.shape[1]

  def body(h_ref, w1_ref, b1_ref, w2_ref, b2_ref, o_ref):
    t = jnp.dot(h_ref[:], w1_ref[:], preferred_element_type=jnp.float32)
    t = jax.nn.gelu(t + b1_ref[:])
    o_ref[:] = jnp.dot(t, w2_ref[:],
                       preferred_element_type=jnp.float32) + b2_ref[:]

  return pl.pallas_call(
      body,
      grid=(_GRID,),
      in_specs=[
          pl.BlockSpec((_BM, _D), lambda i: (i, 0)),
          pl.BlockSpec((_D, ph), lambda i: (0, 0)),
          pl.BlockSpec((1, ph), lambda i: (0, 0)),
          pl.BlockSpec((ph, pd), lambda i: (0, 0)),
          pl.BlockSpec((1, pd), lambda i: (0, 0)),
      ],
      out_specs=pl.BlockSpec((_BM, pd), lambda i: (i, 0)),
      out_shape=jax.ShapeDtypeStruct((_NPAD, pd), jnp.float32),
  )(h, w1, b1.reshape(1, ph), w2, b2.reshape(1, pd))


def kernel(x_drug_v1, x_dis_v1, x_drug_v2, x_dis_v2,
           ei_dd_v1, ei_rev_v1, ei_dd_v2, ei_rev_v2,
           W_rel_dd, W_rel_rev, Ws_drug, Ws_dis,
           Wp1_drug, bp1_drug, Wp2_drug, bp2_drug,
           Wp1_dis, bp1_dis, Wp2_dis, bp2_dis):
  pad_n = lambda x: jnp.pad(x, ((0, _NPAD - _N), (0, 0)))
  hd1, hs1, hd2, hs2 = map(pad_n, (x_drug_v1, x_dis_v1, x_drug_v2, x_dis_v2))

  def pad_e(ei):
    src = jnp.pad(ei[0].astype(jnp.int32), (0, _EPAD - _E))
    dst = jnp.pad(ei[1].astype(jnp.int32), (0, _EPAD - _E),
                  constant_values=_N)
    return src, dst

  s_dd1, d_dd1 = pad_e(ei_dd_v1)
  s_rv1, d_rv1 = pad_e(ei_rev_v1)
  s_dd2, d_dd2 = pad_e(ei_dd_v2)
  s_rv2, d_rv2 = pad_e(ei_rev_v2)

  seg = _make_segsum()
  cntk = _make_counts()

  C_dd1, C_rv1, C_dd2, C_rv2 = cntk(d_dd1, d_rv1, d_dd2, d_rv2)
  S_dd1, S_rv1, S_dd2, S_rv2 = seg(
      hd1, hs1, hd2, hs2,
      s_dd1, s_rv1, s_dd2, s_rv2,
      d_dd1, d_rv1, d_dd2, d_rv2)
  hd1n = _dense_update(hd1, S_rv1, C_rv1, Ws_drug[0], W_rel_rev[0])
  hs1n = _dense_update(hs1, S_dd1, C_dd1, Ws_dis[0], W_rel_dd[0])
  hd2n = _dense_update(hd2, S_rv2, C_rv2, Ws_drug[0], W_rel_rev[0])
  hs2n = _dense_update(hs2, S_dd2, C_dd2, Ws_dis[0], W_rel_dd[0])

  S_dd1, S_rv1, S_dd2, S_rv2 = seg(
      hd1n, hs1n, hd2n, hs2n,
      s_dd1, s_rv1, s_dd2, s_rv2,
      d_dd1, d_rv1, d_dd2, d_rv2)
  hd1f = _dense_update(hd1n, S_rv1, C_rv1, Ws_drug[1], W_rel_rev[1])
  hs1f = _dense_update(hs1n, S_dd1, C_dd1, Ws_dis[1], W_rel_dd[1])
  hd2f = _dense_update(hd2n, S_rv2, C_rv2, Ws_drug[1], W_rel_rev[1])
  hs2f = _dense_update(hs2n, S_dd2, C_dd2, Ws_dis[1], W_rel_dd[1])

  p1d = _project(hd1f, Wp1_drug, bp1_drug, Wp2_drug, bp2_drug)[:_N]
  p1s = _project(hs1f, Wp1_dis, bp1_dis, Wp2_dis, bp2_dis)[:_N]
  p2d = _project(hd2f, Wp1_drug, bp1_drug, Wp2_drug, bp2_drug)[:_N]
  p2s = _project(hs2f, Wp1_dis, bp1_dis, Wp2_dis, bp2_dis)[:_N]
  return (p1d, p1s, p2d, p2s)
